# baseline (device time: 16837 ns/iter reference)
import jax
import jax.numpy as jnp
from jax import lax
from jax.experimental import pallas as pl
from jax.experimental.pallas import tpu as pltpu

N_Y = 2
SUB_SIZES = (16, 112, 128, 128, 128)
K = len(SUB_SIZES)


def kernel(x):
    m, n = x.shape
    half = m // 2
    assert sum(SUB_SIZES) == half
    sub_offs = [sum(SUB_SIZES[:k]) for k in range(K)]

    def body(x_hbm, out_hbm, comm, xv, in_sems, out_sems,
             ysend_sems, yrecv_sems, xsend_sems, xrecv_sems):
        my_x = lax.axis_index("x")
        my_y = lax.axis_index("y")
        nbr_y = (my_x, 1 - my_y)
        nbr_x = (1 - my_x, my_y)

        in_copies = []
        for k in range(K):
            rows = pl.ds(my_x * half + sub_offs[k], SUB_SIZES[k])
            c = pltpu.make_async_copy(x_hbm.at[rows, :], xv.at[rows, :],
                                      in_sems.at[k])
            c.start()
            in_copies.append(c)
        keep_rows = pl.ds((1 - my_x) * half, half)
        c = pltpu.make_async_copy(x_hbm.at[keep_rows, :], xv.at[keep_rows, :],
                                  in_sems.at[K])
        c.start()
        in_copies.append(c)

        barrier = pltpu.get_barrier_semaphore()
        for nbr in (nbr_y, nbr_x):
            pl.semaphore_signal(
                barrier, inc=1, device_id=nbr,
                device_id_type=pl.DeviceIdType.MESH,
            )
        pl.semaphore_wait(barrier, 2)

        gy = my_y * m
        send_base = gy + my_x * half
        keep_base = gy + (1 - my_x) * half
        yrecv_base = (1 - my_y) * m + my_x * half
        xrecv_base = (1 - my_y) * m + (1 - my_x) * half

        out_copies = []

        y_sends = []
        for k in range(K):
            off, sub = sub_offs[k], SUB_SIZES[k]
            rows = pl.ds(send_base + off, sub)
            in_copies[k].wait()
            comm[rows, :] = xv[
                pl.ds(my_x * half + off, sub), :
            ].astype(jnp.bfloat16)
            r = pltpu.make_async_remote_copy(
                src_ref=comm.at[rows, :],
                dst_ref=comm.at[rows, :],
                send_sem=ysend_sems.at[k],
                recv_sem=yrecv_sems.at[k],
                device_id=nbr_y,
                device_id_type=pl.DeviceIdType.MESH,
            )
            r.start()
            y_sends.append(r)

        in_copies[K].wait()
        comm[pl.ds(keep_base, half), :] = xv[keep_rows, :].astype(jnp.bfloat16)
        own_rows = pl.ds(gy, m)
        c = pltpu.make_async_copy(comm.at[own_rows, :], out_hbm.at[own_rows, :],
                                  out_sems.at[2 * K])
        c.start()
        out_copies.append(c)

        x_sends = []
        for k in range(K):
            rows = pl.ds(yrecv_base + sub_offs[k], SUB_SIZES[k])
            recv = pltpu.make_async_remote_copy(
                src_ref=comm.at[rows, :],
                dst_ref=comm.at[rows, :],
                send_sem=ysend_sems.at[k],
                recv_sem=yrecv_sems.at[k],
                device_id=nbr_y,
                device_id_type=pl.DeviceIdType.MESH,
            )
            recv.wait_recv()
            r = pltpu.make_async_remote_copy(
                src_ref=comm.at[rows, :],
                dst_ref=comm.at[rows, :],
                send_sem=xsend_sems.at[k],
                recv_sem=xrecv_sems.at[k],
                device_id=nbr_x,
                device_id_type=pl.DeviceIdType.MESH,
            )
            r.start()
            x_sends.append(r)
            c = pltpu.make_async_copy(comm.at[rows, :], out_hbm.at[rows, :],
                                      out_sems.at[k])
            c.start()
            out_copies.append(c)

        for k in range(K):
            rows = pl.ds(xrecv_base + sub_offs[k], SUB_SIZES[k])
            recv = pltpu.make_async_remote_copy(
                src_ref=comm.at[rows, :],
                dst_ref=comm.at[rows, :],
                send_sem=xsend_sems.at[k],
                recv_sem=xrecv_sems.at[k],
                device_id=nbr_x,
                device_id_type=pl.DeviceIdType.MESH,
            )
            recv.wait_recv()
            c = pltpu.make_async_copy(comm.at[rows, :], out_hbm.at[rows, :],
                                      out_sems.at[K + k])
            c.start()
            out_copies.append(c)

        for r in y_sends:
            r.wait_send()
        for r in x_sends:
            r.wait_send()
        for c in out_copies:
            c.wait()

    return pl.pallas_call(
        body,
        out_shape=jax.ShapeDtypeStruct((N_Y * m, n), jnp.bfloat16),
        in_specs=[pl.BlockSpec(memory_space=pl.ANY)],
        out_specs=pl.BlockSpec(memory_space=pl.ANY),
        scratch_shapes=[
            pltpu.VMEM((N_Y * m, n), jnp.bfloat16),
            pltpu.VMEM((m, n), jnp.float32),
            pltpu.SemaphoreType.DMA((K + 1,)),
            pltpu.SemaphoreType.DMA((2 * K + 1,)),
            pltpu.SemaphoreType.DMA((K,)),
            pltpu.SemaphoreType.DMA((K,)),
            pltpu.SemaphoreType.DMA((K,)),
            pltpu.SemaphoreType.DMA((K,)),
        ],
        compiler_params=pltpu.CompilerParams(collective_id=0),
    )(x)


# device time: 16096 ns/iter; 1.0460x vs baseline; 1.0460x over previous
import jax
import jax.numpy as jnp
from jax import lax
from jax.experimental import pallas as pl
from jax.experimental.pallas import tpu as pltpu

N_Y = 2
SUB_SIZES = (16, 112, 128, 128, 128)
K = len(SUB_SIZES)


def kernel(x):
    m, n = x.shape
    half = m // 2
    assert sum(SUB_SIZES) == half
    sub_offs = [sum(SUB_SIZES[:k]) for k in range(K)]

    def body(x_ref, out_ref, ysend_sems, yrecv_sems, xsend_sems, xrecv_sems):
        my_x = lax.axis_index("x")
        my_y = lax.axis_index("y")
        nbr_y = (my_x, 1 - my_y)
        nbr_x = (1 - my_x, my_y)

        barrier = pltpu.get_barrier_semaphore()
        for nbr in (nbr_y, nbr_x):
            pl.semaphore_signal(
                barrier, inc=1, device_id=nbr,
                device_id_type=pl.DeviceIdType.MESH,
            )
        pl.semaphore_wait(barrier, 2)

        gy = my_y * m
        send_base = gy + my_x * half
        keep_base = gy + (1 - my_x) * half
        yrecv_base = (1 - my_y) * m + my_x * half
        xrecv_base = (1 - my_y) * m + (1 - my_x) * half

        y_sends = []
        for k in range(K):
            off, sub = sub_offs[k], SUB_SIZES[k]
            rows = pl.ds(send_base + off, sub)
            out_ref[rows, :] = x_ref[
                pl.ds(my_x * half + off, sub), :
            ].astype(jnp.bfloat16)
            r = pltpu.make_async_remote_copy(
                src_ref=out_ref.at[rows, :],
                dst_ref=out_ref.at[rows, :],
                send_sem=ysend_sems.at[k],
                recv_sem=yrecv_sems.at[k],
                device_id=nbr_y,
                device_id_type=pl.DeviceIdType.MESH,
            )
            r.start()
            y_sends.append(r)

        out_ref[pl.ds(keep_base, half), :] = x_ref[
            pl.ds((1 - my_x) * half, half), :
        ].astype(jnp.bfloat16)

        x_sends = []
        for k in range(K):
            rows = pl.ds(yrecv_base + sub_offs[k], SUB_SIZES[k])
            recv = pltpu.make_async_remote_copy(
                src_ref=out_ref.at[rows, :],
                dst_ref=out_ref.at[rows, :],
                send_sem=ysend_sems.at[k],
                recv_sem=yrecv_sems.at[k],
                device_id=nbr_y,
                device_id_type=pl.DeviceIdType.MESH,
            )
            recv.wait_recv()
            r = pltpu.make_async_remote_copy(
                src_ref=out_ref.at[rows, :],
                dst_ref=out_ref.at[rows, :],
                send_sem=xsend_sems.at[k],
                recv_sem=xrecv_sems.at[k],
                device_id=nbr_x,
                device_id_type=pl.DeviceIdType.MESH,
            )
            r.start()
            x_sends.append(r)

        for k in range(K):
            rows = pl.ds(xrecv_base + sub_offs[k], SUB_SIZES[k])
            recv = pltpu.make_async_remote_copy(
                src_ref=out_ref.at[rows, :],
                dst_ref=out_ref.at[rows, :],
                send_sem=xsend_sems.at[k],
                recv_sem=xrecv_sems.at[k],
                device_id=nbr_x,
                device_id_type=pl.DeviceIdType.MESH,
            )
            recv.wait_recv()

        for r in y_sends:
            r.wait_send()
        for r in x_sends:
            r.wait_send()

    return pl.pallas_call(
        body,
        out_shape=jax.ShapeDtypeStruct((N_Y * m, n), jnp.bfloat16),
        in_specs=[pl.BlockSpec(memory_space=pltpu.VMEM)],
        out_specs=pl.BlockSpec(memory_space=pltpu.VMEM),
        scratch_shapes=[
            pltpu.SemaphoreType.DMA((K,)),
            pltpu.SemaphoreType.DMA((K,)),
            pltpu.SemaphoreType.DMA((K,)),
            pltpu.SemaphoreType.DMA((K,)),
        ],
        compiler_params=pltpu.CompilerParams(collective_id=0),
    )(x)


# device time: 15459 ns/iter; 1.0891x vs baseline; 1.0412x over previous
import jax
import jax.numpy as jnp
from jax import lax
from jax.experimental import pallas as pl
from jax.experimental.pallas import tpu as pltpu

N_Y = 2
SUB_SIZES = (32,) * 16
K = len(SUB_SIZES)


def kernel(x):
    m, n = x.shape
    half = m // 2
    assert sum(SUB_SIZES) == half
    sub_offs = [sum(SUB_SIZES[:k]) for k in range(K)]

    def body(x_ref, out_ref, ysend_sems, yrecv_sems, xsend_sems, xrecv_sems):
        my_x = lax.axis_index("x")
        my_y = lax.axis_index("y")
        nbr_y = (my_x, 1 - my_y)
        nbr_x = (1 - my_x, my_y)

        barrier = pltpu.get_barrier_semaphore()
        for nbr in (nbr_y, nbr_x):
            pl.semaphore_signal(
                barrier, inc=1, device_id=nbr,
                device_id_type=pl.DeviceIdType.MESH,
            )
        pl.semaphore_wait(barrier, 2)

        gy = my_y * m
        send_base = gy + my_x * half
        keep_base = gy + (1 - my_x) * half
        yrecv_base = (1 - my_y) * m + my_x * half
        xrecv_base = (1 - my_y) * m + (1 - my_x) * half

        y_sends = []
        for k in range(K):
            off, sub = sub_offs[k], SUB_SIZES[k]
            rows = pl.ds(send_base + off, sub)
            out_ref[rows, :] = x_ref[
                pl.ds(my_x * half + off, sub), :
            ].astype(jnp.bfloat16)
            r = pltpu.make_async_remote_copy(
                src_ref=out_ref.at[rows, :],
                dst_ref=out_ref.at[rows, :],
                send_sem=ysend_sems.at[k],
                recv_sem=yrecv_sems.at[k],
                device_id=nbr_y,
                device_id_type=pl.DeviceIdType.MESH,
            )
            r.start()
            y_sends.append(r)

        out_ref[pl.ds(keep_base, half), :] = x_ref[
            pl.ds((1 - my_x) * half, half), :
        ].astype(jnp.bfloat16)

        x_sends = []
        for k in range(K):
            rows = pl.ds(yrecv_base + sub_offs[k], SUB_SIZES[k])
            recv = pltpu.make_async_remote_copy(
                src_ref=out_ref.at[rows, :],
                dst_ref=out_ref.at[rows, :],
                send_sem=ysend_sems.at[k],
                recv_sem=yrecv_sems.at[k],
                device_id=nbr_y,
                device_id_type=pl.DeviceIdType.MESH,
            )
            recv.wait_recv()
            r = pltpu.make_async_remote_copy(
                src_ref=out_ref.at[rows, :],
                dst_ref=out_ref.at[rows, :],
                send_sem=xsend_sems.at[k],
                recv_sem=xrecv_sems.at[k],
                device_id=nbr_x,
                device_id_type=pl.DeviceIdType.MESH,
            )
            r.start()
            x_sends.append(r)

        for k in range(K):
            rows = pl.ds(xrecv_base + sub_offs[k], SUB_SIZES[k])
            recv = pltpu.make_async_remote_copy(
                src_ref=out_ref.at[rows, :],
                dst_ref=out_ref.at[rows, :],
                send_sem=xsend_sems.at[k],
                recv_sem=xrecv_sems.at[k],
                device_id=nbr_x,
                device_id_type=pl.DeviceIdType.MESH,
            )
            recv.wait_recv()

        for r in y_sends:
            r.wait_send()
        for r in x_sends:
            r.wait_send()

    return pl.pallas_call(
        body,
        out_shape=jax.ShapeDtypeStruct((N_Y * m, n), jnp.bfloat16),
        in_specs=[pl.BlockSpec(memory_space=pltpu.VMEM)],
        out_specs=pl.BlockSpec(memory_space=pltpu.VMEM),
        scratch_shapes=[
            pltpu.SemaphoreType.DMA((K,)),
            pltpu.SemaphoreType.DMA((K,)),
            pltpu.SemaphoreType.DMA((K,)),
            pltpu.SemaphoreType.DMA((K,)),
        ],
        compiler_params=pltpu.CompilerParams(collective_id=0),
    )(x)
